# baseline (speedup 1.0000x reference)
import functools

import jax
import jax.numpy as jnp
from jax import lax
from jax.experimental import pallas as pl
from jax.experimental.pallas import tpu as pltpu
from jax.experimental.pallas import tpu_sc as plsc

HIDDEN = 64
B, T = 4096, 200
NC, NS = 2, 16
NW = NC * NS
BT = B // NW
HG, HR, LN = 8, 8, 128

_mesh = plsc.VectorSubcoreMesh(core_axis_name="c", subcore_axis_name="s")


@functools.partial(
    pl.kernel,
    mesh=_mesh,
    compiler_params=pltpu.CompilerParams(
        use_tc_tiling_on_sc=False, needs_layout_passes=False
    ),
    out_type=jax.ShapeDtypeStruct((T, HG, NW, HR, LN), jnp.float32),
    scratch_types=[
        pltpu.VMEM((BT * T,), jnp.int32),
        pltpu.VMEM((2, BT), jnp.int32),
        pltpu.VMEM((2, BT, HIDDEN), jnp.float32),
        pltpu.VMEM((2, HG, HR, LN), jnp.float32),
        pltpu.SemaphoreType.DMA((2,)),
        pltpu.SemaphoreType.DMA((2,)),
    ],
)
def _embed(x_hbm, table_hbm, out5, xs, idxu, rows, tbuf, sem_g, sem_o):
    w = lax.axis_index("s") * NC + lax.axis_index("c")
    pltpu.sync_copy(x_hbm.at[pl.ds(w * BT * T, BT * T)], xs)

    iota = lax.iota(jnp.int32, 16)
    ib = [iota + blk * 16 for blk in range(8)]
    pa = [(iota + blk * 16) * T for blk in range(8)]

    def stage_idx(t, q):
        for blk in range(8):
            v = plsc.load_gather(xs, [pa[blk] + t])
            idxu[q, pl.ds(blk * 16, 16)] = v

    def gather_desc(q):
        return pltpu.make_async_copy(
            table_hbm.at[idxu.at[q]], rows.at[q], sem_g.at[q]
        )

    def out_desc(t, q):
        return pltpu.make_async_copy(tbuf.at[q], out5.at[t, :, w], sem_o.at[q])

    def transpose(q):
        def h_body(hg, _):
            for hr in range(HR):
                h = hg * HR + hr
                colv = jnp.full((16,), h, jnp.int32)
                for blk in range(8):
                    v = plsc.load_gather(rows.at[q], [ib[blk], colv])
                    tbuf[q, hg, hr, pl.ds(blk * 16, 16)] = v
            return 0

        lax.fori_loop(0, HG, h_body, 0)

    stage_idx(0, 0)
    gather_desc(0).start()
    stage_idx(1, 1)
    gather_desc(1).start()
    for q in range(2):
        gather_desc(q).wait()
        transpose(q)
        stage_idx(q + 2, q)
        gather_desc(q).start()
        out_desc(q, q).start()

    def group(g, _):
        for q in range(2):
            t = g * 2 + q
            gather_desc(q).wait()
            out_desc(t - 2, q).wait()
            transpose(q)
            stage_idx(t + 2, q)
            gather_desc(q).start()
            out_desc(t, q).start()
        return 0

    lax.fori_loop(1, T // 2 - 1, group, 0)

    for q, t in ((0, T - 2), (1, T - 1)):
        gather_desc(q).wait()
        out_desc(t - 2, q).wait()
        transpose(q)
        out_desc(t, q).start()
    out_desc(T - 2, 0).wait()
    out_desc(T - 1, 1).wait()


def kernel(x, table):
    xf = x.reshape(-1).astype(jnp.int32)
    out5 = _embed(xf, table)
    y = out5.transpose(2, 4, 0, 1, 3)
    return y.reshape(B, T, HIDDEN)


# speedup vs baseline: 1.7518x; 1.7518x over previous
import functools

import jax
import jax.numpy as jnp
from jax import lax
from jax.experimental import pallas as pl
from jax.experimental.pallas import tpu as pltpu
from jax.experimental.pallas import tpu_sc as plsc

HIDDEN = 64
B, T = 4096, 200
NC, NS = 2, 16
NW = NC * NS
BT = B // NW
HG, HR, LN = 8, 8, 128

_mesh = plsc.VectorSubcoreMesh(core_axis_name="c", subcore_axis_name="s")


@functools.partial(
    pl.kernel,
    mesh=_mesh,
    compiler_params=pltpu.CompilerParams(
        use_tc_tiling_on_sc=False, needs_layout_passes=False
    ),
    out_type=jax.ShapeDtypeStruct((T, HG, NW, HR, LN), jnp.float32),
    scratch_types=[
        pltpu.VMEM((BT * T,), jnp.int32),
        pltpu.VMEM((2, BT), jnp.int32),
        pltpu.VMEM((2, BT, HIDDEN), jnp.float32),
        pltpu.VMEM((2, HG, HR, LN), jnp.float32),
        pltpu.SemaphoreType.DMA((2,)),
        pltpu.SemaphoreType.DMA((2,)),
    ],
)
def _embed(x_hbm, table_hbm, out5, xs, idxu, rows, tbuf, sem_g, sem_o):
    w = lax.axis_index("s") * NC + lax.axis_index("c")
    pltpu.sync_copy(x_hbm.at[pl.ds(w * BT * T, BT * T)], xs)

    iota = lax.iota(jnp.int32, 16)
    ib = [iota + blk * 16 for blk in range(8)]
    pa = [(iota + blk * 16) * T for blk in range(8)]

    def stage_idx(t, q):
        for blk in range(8):
            v = plsc.load_gather(xs, [pa[blk] + t])
            idxu[q, pl.ds(blk * 16, 16)] = v

    def gather_desc(q):
        return pltpu.make_async_copy(
            table_hbm.at[idxu.at[q]], rows.at[q], sem_g.at[q]
        )

    def out_desc(t, q):
        return pltpu.make_async_copy(tbuf.at[q], out5.at[t, :, w], sem_o.at[q])

    def transpose(q):
        @plsc.parallel_loop(0, HIDDEN, 1, unroll=8)
        def _h_body(h):
            hg = h // HR
            hr = h % HR
            colv = jnp.full((16,), h, jnp.int32)
            for blk in range(8):
                v = plsc.load_gather(rows.at[q], [ib[blk], colv])
                tbuf[q, hg, hr, pl.ds(blk * 16, 16)] = v

    stage_idx(0, 0)
    gather_desc(0).start()
    stage_idx(1, 1)
    gather_desc(1).start()
    for q in range(2):
        gather_desc(q).wait()
        transpose(q)
        stage_idx(q + 2, q)
        gather_desc(q).start()
        out_desc(q, q).start()

    def group(g, _):
        for q in range(2):
            t = g * 2 + q
            gather_desc(q).wait()
            out_desc(t - 2, q).wait()
            transpose(q)
            stage_idx(t + 2, q)
            gather_desc(q).start()
            out_desc(t, q).start()
        return 0

    lax.fori_loop(1, T // 2 - 1, group, 0)

    for q, t in ((0, T - 2), (1, T - 1)):
        gather_desc(q).wait()
        out_desc(t - 2, q).wait()
        transpose(q)
        out_desc(t, q).start()
    out_desc(T - 2, 0).wait()
    out_desc(T - 1, 1).wait()


def kernel(x, table):
    xf = x.reshape(-1).astype(jnp.int32)
    out5 = _embed(xf, table)
    y = out5.transpose(2, 4, 0, 1, 3)
    return y.reshape(B, T, HIDDEN)
